# baseline (speedup 1.0000x reference)
import functools

import jax
import jax.numpy as jnp
from jax import lax
from jax.experimental import pallas as pl
from jax.experimental.pallas import tpu as pltpu
from jax.experimental.pallas import tpu_sc as plsc

N = 10000
E = 320000
D = 128
L = 16
PK = 8
NP = N // PK
EP = E // PK
EBLK = 4000
NC = 2
NS = 16
NW = NC * NS
EPW = E // NW
CH = 80
NCHUNK = EPW // CH
NPAD = 10240
NPS = NPAD // NS

_f32 = jnp.float32


def _bd(w):
    return jnp.kron(jnp.eye(PK, dtype=w.dtype), w)


def _bt(b):
    return jnp.tile(b, PK)[None, :]


def _enc_node_body(x_ref, w1_ref, b1_ref, w2_ref, b2_ref, wa_ref, wb_ref,
                   hn_ref, a_ref, b_ref):
    h = jnp.maximum(jnp.dot(x_ref[...], w1_ref[...],
                            preferred_element_type=_f32) + b1_ref[...], 0.0)
    hn = jnp.dot(h, w2_ref[...], preferred_element_type=_f32) + b2_ref[...]
    hn_ref[...] = hn
    a_ref[...] = jnp.dot(hn, wa_ref[...], preferred_element_type=_f32)
    b_ref[...] = jnp.dot(hn, wb_ref[...], preferred_element_type=_f32)


def _enc_edge_step_body(x_ref, ga_ref, gb_ref, we1_ref, be1_ref, we2_ref,
                        be2_ref, w1e_ref, b1_ref, w2_ref, b2_ref,
                        ne_ref, heo_ref):
    h0 = jnp.maximum(jnp.dot(x_ref[...], we1_ref[...],
                             preferred_element_type=_f32) + be1_ref[...], 0.0)
    he = jnp.dot(h0, we2_ref[...], preferred_element_type=_f32) + be2_ref[...]
    h = jnp.maximum(
        ga_ref[...] + gb_ref[...]
        + jnp.dot(he, w1e_ref[...], preferred_element_type=_f32)
        + b1_ref[...], 0.0)
    ne = jnp.dot(h, w2_ref[...], preferred_element_type=_f32) + b2_ref[...]
    ne_ref[...] = ne
    heo_ref[...] = he + ne


def _edge_last_body(ga_ref, gb_ref, he_ref, w1e_ref, b1_ref, w2_ref, b2_ref,
                    ne_ref):
    h = jnp.maximum(
        ga_ref[...] + gb_ref[...]
        + jnp.dot(he_ref[...], w1e_ref[...], preferred_element_type=_f32)
        + b1_ref[...], 0.0)
    ne_ref[...] = jnp.dot(h, w2_ref[...], preferred_element_type=_f32) \
        + b2_ref[...]


def _node_step_body(hn_ref, s_ref, r_ref, w1n_ref, w1s_ref, w1r_ref, b1_ref,
                    w2_ref, b2_ref, wa_ref, wb_ref,
                    hno_ref, a_ref, b_ref):
    aggs = s_ref[0, :NP] + s_ref[1, :NP]
    aggr = r_ref[0, :NP] + r_ref[1, :NP]
    h = jnp.maximum(
        jnp.dot(hn_ref[...], w1n_ref[...], preferred_element_type=_f32)
        + jnp.dot(aggs, w1s_ref[...], preferred_element_type=_f32)
        + jnp.dot(aggr, w1r_ref[...], preferred_element_type=_f32)
        + b1_ref[...], 0.0)
    hn1 = hn_ref[...] + jnp.dot(h, w2_ref[...],
                                preferred_element_type=_f32) + b2_ref[...]
    hno_ref[...] = hn1
    a_ref[...] = jnp.dot(hn1, wa_ref[...], preferred_element_type=_f32)
    b_ref[...] = jnp.dot(hn1, wb_ref[...], preferred_element_type=_f32)


def _node_dec_body(hn_ref, s_ref, r_ref, w1n_ref, w1s_ref, w1r_ref, b1_ref,
                   w2_ref, b2_ref, wd1_ref, bd1_ref, wd2_ref, bd2_ref,
                   aux_ref, out_ref):
    aggs = s_ref[0, :NP] + s_ref[1, :NP]
    aggr = r_ref[0, :NP] + r_ref[1, :NP]
    h = jnp.maximum(
        jnp.dot(hn_ref[...], w1n_ref[...], preferred_element_type=_f32)
        + jnp.dot(aggs, w1s_ref[...], preferred_element_type=_f32)
        + jnp.dot(aggr, w1r_ref[...], preferred_element_type=_f32)
        + b1_ref[...], 0.0)
    hn2 = hn_ref[...] + jnp.dot(h, w2_ref[...],
                                preferred_element_type=_f32) + b2_ref[...]
    hd = jnp.maximum(jnp.dot(hn2, wd1_ref[...],
                             preferred_element_type=_f32) + bd1_ref[...], 0.0)
    out_ref[...] = jnp.dot(hd, wd2_ref[...], preferred_element_type=_f32) \
        + bd2_ref[...] + aux_ref[...]


def _full(shape):
    return pl.BlockSpec(shape, lambda: tuple(0 for _ in shape))


def _single(body, out_shapes, args):
    return pl.pallas_call(
        body,
        grid=(),
        in_specs=[_full(a.shape) for a in args],
        out_specs=[_full(s.shape) for s in out_shapes],
        out_shape=out_shapes,
    )(*args)


def _edge_grid(body, n_out, rows_args, wb_args):
    row = pl.BlockSpec((EBLK, 128), lambda i: (i, 0))

    def wb_spec(a):
        return pl.BlockSpec(a.shape, lambda i: tuple(0 for _ in a.shape))

    out_shape = [jax.ShapeDtypeStruct((EP, 128), _f32)] * n_out
    return pl.pallas_call(
        body,
        grid=(EP // EBLK,),
        in_specs=[row] * len(rows_args) + [wb_spec(a) for a in wb_args],
        out_specs=[row] * n_out,
        out_shape=out_shape,
        compiler_params=pltpu.CompilerParams(
            dimension_semantics=("parallel",)),
    )(*rows_args, *wb_args)


def _sc_gather(a, b, s2, r2):
    mesh = plsc.VectorSubcoreMesh(core_axis_name="c", subcore_axis_name="s")

    @functools.partial(
        pl.kernel,
        out_type=(jax.ShapeDtypeStruct((E, 16), _f32),
                  jax.ShapeDtypeStruct((E, 16), _f32)),
        mesh=mesh,
        scratch_types=[
            pltpu.VMEM((NCHUNK, CH), jnp.int32),
            pltpu.VMEM((NCHUNK, CH), jnp.int32),
            pltpu.VMEM((CH, 16), _f32),
            pltpu.VMEM((CH, 16), _f32),
            pltpu.VMEM((CH, 16), _f32),
            pltpu.VMEM((CH, 16), _f32),
            pltpu.SemaphoreType.DMA,
            pltpu.SemaphoreType.DMA,
            pltpu.SemaphoreType.DMA,
            pltpu.SemaphoreType.DMA,
        ],
        compiler_params=pltpu.CompilerParams(use_tc_tiling_on_sc=False),
    )
    def k(a_hbm, b_hbm, s_hbm, r_hbm, ga_hbm, gb_hbm, sidx, ridx,
          ra0, rb0, ra1, rb1, sa0, sb0, sa1, sb1):
        wid = lax.axis_index("s") * NC + lax.axis_index("c")
        pltpu.sync_copy(s_hbm.at[wid], sidx)
        pltpu.sync_copy(r_hbm.at[wid], ridx)
        base = wid * EPW
        slots = ((ra0, rb0, sa0, sb0), (ra1, rb1, sa1, sb1))

        def issue(j, slot):
            ra, rb, sa, sb = slot
            pltpu.async_copy(a_hbm.at[sidx.at[j]], ra, sa)
            pltpu.async_copy(b_hbm.at[ridx.at[j]], rb, sb)

        def drain_store(j, slot):
            ra, rb, sa, sb = slot
            pltpu.make_async_copy(a_hbm.at[sidx.at[j]], ra, sa).wait()
            pltpu.make_async_copy(b_hbm.at[ridx.at[j]], rb, sb).wait()
            pltpu.sync_copy(ra, ga_hbm.at[pl.ds(base + j * CH, CH)])
            pltpu.sync_copy(rb, gb_hbm.at[pl.ds(base + j * CH, CH)])

        issue(0, slots[0])

        @pl.loop(0, (NCHUNK - 1) // 2)
        def _(t):
            issue(2 * t + 1, slots[1])
            drain_store(2 * t, slots[0])
            issue(2 * t + 2, slots[0])
            drain_store(2 * t + 1, slots[1])

        drain_store(NCHUNK - 1, slots[0])

    return k(a, b, s2, r2)


def _sc_scatter(ne, s2, r2):
    mesh = plsc.VectorSubcoreMesh(core_axis_name="c", subcore_axis_name="s")

    @functools.partial(
        pl.kernel,
        out_type=(jax.ShapeDtypeStruct((NC, NPAD, 16), _f32),
                  jax.ShapeDtypeStruct((NC, NPAD, 16), _f32)),
        mesh=mesh,
        scratch_types=[
            pltpu.VMEM((NCHUNK, CH), jnp.int32),
            pltpu.VMEM((NCHUNK, CH), jnp.int32),
            pltpu.VMEM((CH, 16), _f32),
            pltpu.VMEM((CH, 16), _f32),
            pltpu.VMEM((128, 16), _f32),
            pltpu.VMEM_SHARED((NPAD, 16), _f32),
            pltpu.VMEM_SHARED((NPAD, 16), _f32),
            pltpu.SemaphoreType.DMA,
            pltpu.SemaphoreType.DMA,
            pltpu.SemaphoreType.DMA,
            pltpu.SemaphoreType.DMA,
        ],
        compiler_params=pltpu.CompilerParams(use_tc_tiling_on_sc=False),
    )
    def k(ne_hbm, s_hbm, r_hbm, outs_hbm, outr_hbm, sidx, ridx, rows0, rows1,
          zbuf, accs, accr, sl0, sl1, sadd_s, sadd_r):
        cid = lax.axis_index("c")
        sid = lax.axis_index("s")
        wid = sid * NC + cid

        @pl.loop(0, 128)
        def _(i):
            zbuf[i, :] = jnp.zeros((16,), _f32)

        @pl.loop(0, NPS // 128)
        def _(t):
            pltpu.sync_copy(zbuf, accs.at[pl.ds(sid * NPS + t * 128, 128)])
            pltpu.sync_copy(zbuf, accr.at[pl.ds(sid * NPS + t * 128, 128)])

        pltpu.sync_copy(s_hbm.at[pl.ds(wid * NCHUNK, NCHUNK)], sidx)
        pltpu.sync_copy(r_hbm.at[pl.ds(wid * NCHUNK, NCHUNK)], ridx)
        plsc.subcore_barrier()
        base = wid * EPW
        slots = ((rows0, sl0), (rows1, sl1))

        def issue(j, slot):
            rows, sl = slot
            pltpu.async_copy(ne_hbm.at[pl.ds(base + j * CH, CH)], rows, sl)

        def drain_add(j, slot):
            rows, sl = slot
            pltpu.make_async_copy(
                ne_hbm.at[pl.ds(base + j * CH, CH)], rows, sl).wait()
            ca = pltpu.async_copy(rows, accs.at[sidx.at[j]], sadd_s, add=True)
            cb = pltpu.async_copy(rows, accr.at[ridx.at[j]], sadd_r, add=True)
            ca.wait()
            cb.wait()

        issue(0, slots[0])

        @pl.loop(0, (NCHUNK - 1) // 2)
        def _(t):
            issue(2 * t + 1, slots[1])
            drain_add(2 * t, slots[0])
            issue(2 * t + 2, slots[0])
            drain_add(2 * t + 1, slots[1])

        drain_add(NCHUNK - 1, slots[0])

        plsc.subcore_barrier()
        pltpu.sync_copy(accs.at[pl.ds(sid * NPS, NPS)],
                        outs_hbm.at[cid, pl.ds(sid * NPS, NPS)])
        pltpu.sync_copy(accr.at[pl.ds(sid * NPS, NPS)],
                        outr_hbm.at[cid, pl.ds(sid * NPS, NPS)])

    return k(ne, s2, r2)


def kernel(nodes, edges, senders, receivers, aux_data, params):
    (we1, be1), (we2, be2) = params['enc_node']
    (wee1, bee1), (wee2, bee2) = params['enc_edge']
    pe = [params['proc_edge_0'], params['proc_edge_1']]
    pn = [params['proc_node_0'], params['proc_node_1']]
    (wd1, bd1), (wd2, bd2) = params['dec_node']

    pe_e = [p[0][0][0:L] for p in pe]
    pe_s = [p[0][0][L:2 * L] for p in pe]
    pe_r = [p[0][0][2 * L:] for p in pe]
    pn_n = [p[0][0][0:L] for p in pn]
    pn_s = [p[0][0][L:2 * L] for p in pn]
    pn_r = [p[0][0][2 * L:] for p in pn]

    s2 = senders.reshape(E // CH, CH)
    r2 = receivers.reshape(E // CH, CH)
    s3 = senders.reshape(NW, NCHUNK, CH)
    r3 = receivers.reshape(NW, NCHUNK, CH)
    nodes_p = nodes.reshape(NP, PK * D)
    edges_p = edges.reshape(EP, 128)
    aux_p = aux_data.reshape(NP, PK * D)

    hn_p, a0_p, b0_p = _single(
        _enc_node_body,
        [jax.ShapeDtypeStruct((NP, 128), _f32)] * 3,
        (nodes_p, _bd(we1), _bt(be1), _bd(we2), _bt(be2),
         _bd(pe_s[0]), _bd(pe_r[0])),
    )
    a_p, b_p = a0_p, b0_p
    he_p = None
    for step in range(2):
        ga, gb = _sc_gather(a_p.reshape(N, 16), b_p.reshape(N, 16), s3, r3)
        ga_p = ga.reshape(EP, 128)
        gb_p = gb.reshape(EP, 128)
        (w1, b1), (w2, b2) = pe[step]
        wb = (_bd(pe_e[step]), _bt(b1), _bd(w2), _bt(b2))
        if step == 0:
            ne_p, he_p = _edge_grid(
                _enc_edge_step_body, 2, (edges_p, ga_p, gb_p),
                (_bd(wee1), _bt(bee1), _bd(wee2), _bt(bee2)) + wb)
        else:
            (ne_p,) = _edge_grid(_edge_last_body, 1, (ga_p, gb_p, he_p), wb)
        aggs, aggr = _sc_scatter(ne_p.reshape(E, 16), s2, r2)
        (nw1, nb1), (nw2, nb2) = pn[step]
        s_parts = aggs.reshape(NC, NPAD // PK, 128)
        r_parts = aggr.reshape(NC, NPAD // PK, 128)
        if step == 0:
            hn_p, a_p, b_p = _single(
                _node_step_body,
                [jax.ShapeDtypeStruct((NP, 128), _f32)] * 3,
                (hn_p, s_parts, r_parts,
                 _bd(pn_n[0]), _bd(pn_s[0]), _bd(pn_r[0]), _bt(nb1),
                 _bd(nw2), _bt(nb2), _bd(pe_s[1]), _bd(pe_r[1])),
            )
        else:
            (out_p,) = _single(
                _node_dec_body,
                [jax.ShapeDtypeStruct((NP, PK * D), _f32)],
                (hn_p, s_parts, r_parts,
                 _bd(pn_n[1]), _bd(pn_s[1]), _bd(pn_r[1]), _bt(nb1),
                 _bd(nw2), _bt(nb2),
                 _bd(wd1), _bt(bd1), _bd(wd2), _bt(bd2), aux_p),
            )
    return out_p.reshape(N, D)


# speedup vs baseline: 1.1697x; 1.1697x over previous
import functools

import jax
import jax.numpy as jnp
from jax import lax
from jax.experimental import pallas as pl
from jax.experimental.pallas import tpu as pltpu
from jax.experimental.pallas import tpu_sc as plsc

N = 10000
E = 320000
D = 128
L = 16
PK = 8
NP = N // PK
EP = E // PK
EBLK = 4000
NC = 2
NS = 16
NW = NC * NS
EPW = E // NW
CH = 80
NCHUNK = EPW // CH
NPAD = 10240
NPS = NPAD // NS

_f32 = jnp.float32


def _bd(w):
    return jnp.kron(jnp.eye(PK, dtype=w.dtype), w)


def _bt(b):
    return jnp.tile(b, PK)[None, :]


def _enc_node_body(x_ref, w1_ref, b1_ref, w2_ref, b2_ref, wa_ref, wb_ref,
                   hn_ref, a_ref, b_ref):
    h = jnp.maximum(jnp.dot(x_ref[...], w1_ref[...],
                            preferred_element_type=_f32) + b1_ref[...], 0.0)
    hn = jnp.dot(h, w2_ref[...], preferred_element_type=_f32) + b2_ref[...]
    hn_ref[...] = hn
    a_ref[...] = jnp.dot(hn, wa_ref[...], preferred_element_type=_f32)
    b_ref[...] = jnp.dot(hn, wb_ref[...], preferred_element_type=_f32)


def _enc_edge_step_body(x_ref, g_ref, we1_ref, be1_ref, we2_ref, be2_ref,
                        w1e_ref, b1_ref, w2_ref, b2_ref, ne_ref, heo_ref):
    h0 = jnp.maximum(jnp.dot(x_ref[...], we1_ref[...],
                             preferred_element_type=_f32) + be1_ref[...], 0.0)
    he = jnp.dot(h0, we2_ref[...], preferred_element_type=_f32) + be2_ref[...]
    h = jnp.maximum(
        g_ref[...]
        + jnp.dot(he, w1e_ref[...], preferred_element_type=_f32)
        + b1_ref[...], 0.0)
    ne = jnp.dot(h, w2_ref[...], preferred_element_type=_f32) + b2_ref[...]
    ne_ref[...] = ne
    heo_ref[...] = he + ne


def _edge_last_body(g_ref, he_ref, w1e_ref, b1_ref, w2_ref, b2_ref, ne_ref):
    h = jnp.maximum(
        g_ref[...]
        + jnp.dot(he_ref[...], w1e_ref[...], preferred_element_type=_f32)
        + b1_ref[...], 0.0)
    ne_ref[...] = jnp.dot(h, w2_ref[...], preferred_element_type=_f32) \
        + b2_ref[...]


def _node_step_body(hn_ref, s_ref, r_ref, w1n_ref, w1s_ref, w1r_ref, b1_ref,
                    w2_ref, b2_ref, wa_ref, wb_ref,
                    hno_ref, a_ref, b_ref):
    aggs = s_ref[0, :NP] + s_ref[1, :NP]
    aggr = r_ref[0, :NP] + r_ref[1, :NP]
    h = jnp.maximum(
        jnp.dot(hn_ref[...], w1n_ref[...], preferred_element_type=_f32)
        + jnp.dot(aggs, w1s_ref[...], preferred_element_type=_f32)
        + jnp.dot(aggr, w1r_ref[...], preferred_element_type=_f32)
        + b1_ref[...], 0.0)
    hn1 = hn_ref[...] + jnp.dot(h, w2_ref[...],
                                preferred_element_type=_f32) + b2_ref[...]
    hno_ref[...] = hn1
    a_ref[...] = jnp.dot(hn1, wa_ref[...], preferred_element_type=_f32)
    b_ref[...] = jnp.dot(hn1, wb_ref[...], preferred_element_type=_f32)


def _node_dec_body(hn_ref, s_ref, r_ref, w1n_ref, w1s_ref, w1r_ref, b1_ref,
                   w2_ref, b2_ref, wd1_ref, bd1_ref, wd2_ref, bd2_ref,
                   aux_ref, out_ref):
    aggs = s_ref[0, :NP] + s_ref[1, :NP]
    aggr = r_ref[0, :NP] + r_ref[1, :NP]
    h = jnp.maximum(
        jnp.dot(hn_ref[...], w1n_ref[...], preferred_element_type=_f32)
        + jnp.dot(aggs, w1s_ref[...], preferred_element_type=_f32)
        + jnp.dot(aggr, w1r_ref[...], preferred_element_type=_f32)
        + b1_ref[...], 0.0)
    hn2 = hn_ref[...] + jnp.dot(h, w2_ref[...],
                                preferred_element_type=_f32) + b2_ref[...]
    hd = jnp.maximum(jnp.dot(hn2, wd1_ref[...],
                             preferred_element_type=_f32) + bd1_ref[...], 0.0)
    out_ref[...] = jnp.dot(hd, wd2_ref[...], preferred_element_type=_f32) \
        + bd2_ref[...] + aux_ref[...]


def _full(shape):
    return pl.BlockSpec(shape, lambda: tuple(0 for _ in shape))


def _single(body, out_shapes, args):
    return pl.pallas_call(
        body,
        grid=(),
        in_specs=[_full(a.shape) for a in args],
        out_specs=[_full(s.shape) for s in out_shapes],
        out_shape=out_shapes,
    )(*args)


def _edge_grid(body, n_out, rows_args, wb_args):
    row = pl.BlockSpec((EBLK, 128), lambda i: (i, 0))

    def wb_spec(a):
        return pl.BlockSpec(a.shape, lambda i: tuple(0 for _ in a.shape))

    out_shape = [jax.ShapeDtypeStruct((EP, 128), _f32)] * n_out
    return pl.pallas_call(
        body,
        grid=(EP // EBLK,),
        in_specs=[row] * len(rows_args) + [wb_spec(a) for a in wb_args],
        out_specs=[row] * n_out,
        out_shape=out_shape,
        compiler_params=pltpu.CompilerParams(
            dimension_semantics=("parallel",)),
    )(*rows_args, *wb_args)


NTS = N // NS


def _sc_gather(a, b, s2, r2):
    mesh = plsc.VectorSubcoreMesh(core_axis_name="c", subcore_axis_name="s")

    @functools.partial(
        pl.kernel,
        out_type=jax.ShapeDtypeStruct((EP, 128), _f32),
        mesh=mesh,
        scratch_types=[
            pltpu.VMEM((NCHUNK, CH), jnp.int32),
            pltpu.VMEM((NCHUNK, CH), jnp.int32),
            pltpu.VMEM((CH, 16), _f32),
            pltpu.VMEM((CH, 16), _f32),
            pltpu.VMEM((CH, 16), _f32),
            pltpu.VMEM((CH, 16), _f32),
            pltpu.VMEM((CH // PK, 128), _f32),
            pltpu.VMEM_SHARED((N, 16), _f32),
            pltpu.VMEM_SHARED((N, 16), _f32),
            pltpu.SemaphoreType.DMA,
            pltpu.SemaphoreType.DMA,
            pltpu.SemaphoreType.DMA,
            pltpu.SemaphoreType.DMA,
        ],
        compiler_params=pltpu.CompilerParams(use_tc_tiling_on_sc=False),
    )
    def k(a_hbm, b_hbm, s_hbm, r_hbm, g_hbm, sidx, ridx,
          ra0, rb0, ra1, rb1, ro, a_sp, b_sp, sa0, sb0, sa1, sb1):
        sid = lax.axis_index("s")
        wid = sid * NC + lax.axis_index("c")
        pltpu.sync_copy(a_hbm.at[pl.ds(sid * NTS, NTS)],
                        a_sp.at[pl.ds(sid * NTS, NTS)])
        pltpu.sync_copy(b_hbm.at[pl.ds(sid * NTS, NTS)],
                        b_sp.at[pl.ds(sid * NTS, NTS)])
        pltpu.sync_copy(s_hbm.at[wid], sidx)
        pltpu.sync_copy(r_hbm.at[wid], ridx)
        plsc.subcore_barrier()
        pbase = wid * (EPW // PK)
        cp = CH // PK
        slots = ((ra0, rb0, sa0, sb0), (ra1, rb1, sa1, sb1))

        def issue(j, slot):
            ra, rb, sa, sb = slot
            pltpu.async_copy(a_sp.at[sidx.at[j]], ra, sa)
            pltpu.async_copy(b_sp.at[ridx.at[j]], rb, sb)

        def drain_store(j, slot):
            ra, rb, sa, sb = slot
            pltpu.make_async_copy(a_sp.at[sidx.at[j]], ra, sa).wait()
            pltpu.make_async_copy(b_sp.at[ridx.at[j]], rb, sb).wait()

            @pl.loop(0, cp)
            def _(i2):
                for kk in range(PK):
                    ro[i2, pl.ds(16 * kk, 16)] = (
                        ra[i2 * PK + kk, :] + rb[i2 * PK + kk, :])

            pltpu.sync_copy(ro, g_hbm.at[pl.ds(pbase + j * cp, cp)])

        issue(0, slots[0])

        @pl.loop(0, (NCHUNK - 1) // 2)
        def _(t):
            issue(2 * t + 1, slots[1])
            drain_store(2 * t, slots[0])
            issue(2 * t + 2, slots[0])
            drain_store(2 * t + 1, slots[1])

        drain_store(NCHUNK - 1, slots[0])

    return k(a, b, s2, r2)


def _sc_scatter(ne, s2, r2):
    mesh = plsc.VectorSubcoreMesh(core_axis_name="c", subcore_axis_name="s")

    @functools.partial(
        pl.kernel,
        out_type=(jax.ShapeDtypeStruct((NC, NPAD, 16), _f32),
                  jax.ShapeDtypeStruct((NC, NPAD, 16), _f32)),
        mesh=mesh,
        scratch_types=[
            pltpu.VMEM((NCHUNK, CH), jnp.int32),
            pltpu.VMEM((NCHUNK, CH), jnp.int32),
            pltpu.VMEM((CH, 16), _f32),
            pltpu.VMEM((CH, 16), _f32),
            pltpu.VMEM((128, 16), _f32),
            pltpu.VMEM_SHARED((NPAD, 16), _f32),
            pltpu.VMEM_SHARED((NPAD, 16), _f32),
            pltpu.SemaphoreType.DMA,
            pltpu.SemaphoreType.DMA,
            pltpu.SemaphoreType.DMA,
            pltpu.SemaphoreType.DMA,
        ],
        compiler_params=pltpu.CompilerParams(use_tc_tiling_on_sc=False),
    )
    def k(ne_hbm, s_hbm, r_hbm, outs_hbm, outr_hbm, sidx, ridx, rows0, rows1,
          zbuf, accs, accr, sl0, sl1, sadd_s, sadd_r):
        cid = lax.axis_index("c")
        sid = lax.axis_index("s")
        wid = sid * NC + cid

        @pl.loop(0, 128)
        def _(i):
            zbuf[i, :] = jnp.zeros((16,), _f32)

        @pl.loop(0, NPS // 128)
        def _(t):
            pltpu.sync_copy(zbuf, accs.at[pl.ds(sid * NPS + t * 128, 128)])
            pltpu.sync_copy(zbuf, accr.at[pl.ds(sid * NPS + t * 128, 128)])

        pltpu.sync_copy(s_hbm.at[pl.ds(wid * NCHUNK, NCHUNK)], sidx)
        pltpu.sync_copy(r_hbm.at[pl.ds(wid * NCHUNK, NCHUNK)], ridx)
        plsc.subcore_barrier()
        base = wid * EPW
        slots = ((rows0, sl0), (rows1, sl1))

        def issue(j, slot):
            rows, sl = slot
            pltpu.async_copy(ne_hbm.at[pl.ds(base + j * CH, CH)], rows, sl)

        def drain_add(j, slot):
            rows, sl = slot
            pltpu.make_async_copy(
                ne_hbm.at[pl.ds(base + j * CH, CH)], rows, sl).wait()
            ca = pltpu.async_copy(rows, accs.at[sidx.at[j]], sadd_s, add=True)
            cb = pltpu.async_copy(rows, accr.at[ridx.at[j]], sadd_r, add=True)
            ca.wait()
            cb.wait()

        issue(0, slots[0])

        @pl.loop(0, (NCHUNK - 1) // 2)
        def _(t):
            issue(2 * t + 1, slots[1])
            drain_add(2 * t, slots[0])
            issue(2 * t + 2, slots[0])
            drain_add(2 * t + 1, slots[1])

        drain_add(NCHUNK - 1, slots[0])

        plsc.subcore_barrier()
        pltpu.sync_copy(accs.at[pl.ds(sid * NPS, NPS)],
                        outs_hbm.at[cid, pl.ds(sid * NPS, NPS)])
        pltpu.sync_copy(accr.at[pl.ds(sid * NPS, NPS)],
                        outr_hbm.at[cid, pl.ds(sid * NPS, NPS)])

    return k(ne, s2, r2)


def kernel(nodes, edges, senders, receivers, aux_data, params):
    (we1, be1), (we2, be2) = params['enc_node']
    (wee1, bee1), (wee2, bee2) = params['enc_edge']
    pe = [params['proc_edge_0'], params['proc_edge_1']]
    pn = [params['proc_node_0'], params['proc_node_1']]
    (wd1, bd1), (wd2, bd2) = params['dec_node']

    pe_e = [p[0][0][0:L] for p in pe]
    pe_s = [p[0][0][L:2 * L] for p in pe]
    pe_r = [p[0][0][2 * L:] for p in pe]
    pn_n = [p[0][0][0:L] for p in pn]
    pn_s = [p[0][0][L:2 * L] for p in pn]
    pn_r = [p[0][0][2 * L:] for p in pn]

    s2 = senders.reshape(E // CH, CH)
    r2 = receivers.reshape(E // CH, CH)
    s3 = senders.reshape(NW, NCHUNK, CH)
    r3 = receivers.reshape(NW, NCHUNK, CH)
    nodes_p = nodes.reshape(NP, PK * D)
    edges_p = edges.reshape(EP, 128)
    aux_p = aux_data.reshape(NP, PK * D)

    hn_p, a0_p, b0_p = _single(
        _enc_node_body,
        [jax.ShapeDtypeStruct((NP, 128), _f32)] * 3,
        (nodes_p, _bd(we1), _bt(be1), _bd(we2), _bt(be2),
         _bd(pe_s[0]), _bd(pe_r[0])),
    )
    a_p, b_p = a0_p, b0_p
    he_p = None
    for step in range(2):
        g = _sc_gather(a_p.reshape(N, 16), b_p.reshape(N, 16), s3, r3)
        (w1, b1), (w2, b2) = pe[step]
        wb = (_bd(pe_e[step]), _bt(b1), _bd(w2), _bt(b2))
        if step == 0:
            ne_p, he_p = _edge_grid(
                _enc_edge_step_body, 2, (edges_p, g),
                (_bd(wee1), _bt(bee1), _bd(wee2), _bt(bee2)) + wb)
        else:
            (ne_p,) = _edge_grid(_edge_last_body, 1, (g, he_p), wb)
        aggs, aggr = _sc_scatter(ne_p.reshape(E, 16), s2, r2)
        (nw1, nb1), (nw2, nb2) = pn[step]
        s_parts = aggs.reshape(NC, NPAD // PK, 128)
        r_parts = aggr.reshape(NC, NPAD // PK, 128)
        if step == 0:
            hn_p, a_p, b_p = _single(
                _node_step_body,
                [jax.ShapeDtypeStruct((NP, 128), _f32)] * 3,
                (hn_p, s_parts, r_parts,
                 _bd(pn_n[0]), _bd(pn_s[0]), _bd(pn_r[0]), _bt(nb1),
                 _bd(nw2), _bt(nb2), _bd(pe_s[1]), _bd(pe_r[1])),
            )
        else:
            (out_p,) = _single(
                _node_dec_body,
                [jax.ShapeDtypeStruct((NP, PK * D), _f32)],
                (hn_p, s_parts, r_parts,
                 _bd(pn_n[1]), _bd(pn_s[1]), _bd(pn_r[1]), _bt(nb1),
                 _bd(nw2), _bt(nb2),
                 _bd(wd1), _bt(bd1), _bd(wd2), _bt(bd2), aux_p),
            )
    return out_p.reshape(N, D)


# speedup vs baseline: 1.1969x; 1.0232x over previous
import functools

import jax
import jax.numpy as jnp
from jax import lax
from jax.experimental import pallas as pl
from jax.experimental.pallas import tpu as pltpu
from jax.experimental.pallas import tpu_sc as plsc

N = 10000
E = 320000
D = 128
L = 16
PK = 8
NP = N // PK
EP = E // PK
EBLK = 4000
NC = 2
NS = 16
NW = NC * NS
EPW = E // NW
CH = 80
NCHUNK = EPW // CH
NPAD = 10240
NPS = NPAD // NS

_f32 = jnp.float32


def _bd(w):
    return jnp.kron(jnp.eye(PK, dtype=w.dtype), w)


def _bt(b):
    return jnp.tile(b, PK)[None, :]


def _enc_node_body(x_ref, w1_ref, b1_ref, w2_ref, b2_ref, wa_ref, wb_ref,
                   hn_ref, a_ref, b_ref):
    h = jnp.maximum(jnp.dot(x_ref[...], w1_ref[...],
                            preferred_element_type=_f32) + b1_ref[...], 0.0)
    hn = jnp.dot(h, w2_ref[...], preferred_element_type=_f32) + b2_ref[...]
    hn_ref[...] = hn
    a_ref[...] = jnp.dot(hn, wa_ref[...], preferred_element_type=_f32)
    b_ref[...] = jnp.dot(hn, wb_ref[...], preferred_element_type=_f32)


def _enc_edge_step_body(x_ref, g_ref, we1_ref, be1_ref, we2_ref, be2_ref,
                        w1e_ref, b1_ref, w2_ref, b2_ref, ne_ref, heo_ref):
    xp = jnp.concatenate(
        [x_ref[:, q * EBLK:(q + 1) * EBLK].T for q in range(PK)], axis=1)
    h0 = jnp.maximum(jnp.dot(xp, we1_ref[...],
                             preferred_element_type=_f32) + be1_ref[...], 0.0)
    he = jnp.dot(h0, we2_ref[...], preferred_element_type=_f32) + be2_ref[...]
    h = jnp.maximum(
        g_ref[...]
        + jnp.dot(he, w1e_ref[...], preferred_element_type=_f32)
        + b1_ref[...], 0.0)
    ne = jnp.dot(h, w2_ref[...], preferred_element_type=_f32) + b2_ref[...]
    ne_ref[...] = ne
    heo_ref[...] = he + ne


def _edge_last_body(g_ref, he_ref, w1e_ref, b1_ref, w2_ref, b2_ref, ne_ref):
    h = jnp.maximum(
        g_ref[...]
        + jnp.dot(he_ref[...], w1e_ref[...], preferred_element_type=_f32)
        + b1_ref[...], 0.0)
    ne_ref[...] = jnp.dot(h, w2_ref[...], preferred_element_type=_f32) \
        + b2_ref[...]


def _node_step_body(hn_ref, s_ref, r_ref, w1n_ref, w1s_ref, w1r_ref, b1_ref,
                    w2_ref, b2_ref, wa_ref, wb_ref,
                    hno_ref, a_ref, b_ref):
    aggs = s_ref[0, :NP] + s_ref[1, :NP]
    aggr = r_ref[0, :NP] + r_ref[1, :NP]
    h = jnp.maximum(
        jnp.dot(hn_ref[...], w1n_ref[...], preferred_element_type=_f32)
        + jnp.dot(aggs, w1s_ref[...], preferred_element_type=_f32)
        + jnp.dot(aggr, w1r_ref[...], preferred_element_type=_f32)
        + b1_ref[...], 0.0)
    hn1 = hn_ref[...] + jnp.dot(h, w2_ref[...],
                                preferred_element_type=_f32) + b2_ref[...]
    hno_ref[...] = hn1
    a_ref[...] = jnp.dot(hn1, wa_ref[...], preferred_element_type=_f32)
    b_ref[...] = jnp.dot(hn1, wb_ref[...], preferred_element_type=_f32)


def _node_dec_body(hn_ref, s_ref, r_ref, w1n_ref, w1s_ref, w1r_ref, b1_ref,
                   w2_ref, b2_ref, wd1_ref, bd1_ref, wd2_ref, bd2_ref,
                   aux_ref, out_ref):
    aggs = s_ref[0, :NP] + s_ref[1, :NP]
    aggr = r_ref[0, :NP] + r_ref[1, :NP]
    h = jnp.maximum(
        jnp.dot(hn_ref[...], w1n_ref[...], preferred_element_type=_f32)
        + jnp.dot(aggs, w1s_ref[...], preferred_element_type=_f32)
        + jnp.dot(aggr, w1r_ref[...], preferred_element_type=_f32)
        + b1_ref[...], 0.0)
    hn2 = hn_ref[...] + jnp.dot(h, w2_ref[...],
                                preferred_element_type=_f32) + b2_ref[...]
    hd = jnp.maximum(jnp.dot(hn2, wd1_ref[...],
                             preferred_element_type=_f32) + bd1_ref[...], 0.0)
    out_ref[...] = jnp.dot(hd, wd2_ref[...], preferred_element_type=_f32) \
        + bd2_ref[...] + aux_ref[...]


def _full(shape):
    return pl.BlockSpec(shape, lambda: tuple(0 for _ in shape))


def _single(body, out_shapes, args):
    return pl.pallas_call(
        body,
        grid=(),
        in_specs=[_full(a.shape) for a in args],
        out_specs=[_full(s.shape) for s in out_shapes],
        out_shape=out_shapes,
    )(*args)


def _edge_grid(body, n_out, rows_args, wb_args, first_spec=None):
    row = pl.BlockSpec((EBLK, 128), lambda i: (i, 0))
    row_specs = [row] * len(rows_args)
    if first_spec is not None:
        row_specs[0] = first_spec

    def wb_spec(a):
        return pl.BlockSpec(a.shape, lambda i: tuple(0 for _ in a.shape))

    out_shape = [jax.ShapeDtypeStruct((EP, 128), _f32)] * n_out
    return pl.pallas_call(
        body,
        grid=(EP // EBLK,),
        in_specs=row_specs + [wb_spec(a) for a in wb_args],
        out_specs=[row] * n_out,
        out_shape=out_shape,
        compiler_params=pltpu.CompilerParams(
            dimension_semantics=("parallel",)),
    )(*rows_args, *wb_args)


NTS = N // NS


def _sc_gather(a, b, s2, r2):
    mesh = plsc.VectorSubcoreMesh(core_axis_name="c", subcore_axis_name="s")

    @functools.partial(
        pl.kernel,
        out_type=jax.ShapeDtypeStruct((EP, 128), _f32),
        mesh=mesh,
        scratch_types=[
            pltpu.VMEM((NCHUNK, CH), jnp.int32),
            pltpu.VMEM((NCHUNK, CH), jnp.int32),
            pltpu.VMEM((CH, 16), _f32),
            pltpu.VMEM((CH, 16), _f32),
            pltpu.VMEM((CH, 16), _f32),
            pltpu.VMEM((CH, 16), _f32),
            pltpu.VMEM((CH // PK, 128), _f32),
            pltpu.VMEM_SHARED((N, 16), _f32),
            pltpu.VMEM_SHARED((N, 16), _f32),
            pltpu.SemaphoreType.DMA,
            pltpu.SemaphoreType.DMA,
            pltpu.SemaphoreType.DMA,
            pltpu.SemaphoreType.DMA,
        ],
        compiler_params=pltpu.CompilerParams(use_tc_tiling_on_sc=False),
    )
    def k(a_hbm, b_hbm, s_hbm, r_hbm, g_hbm, sidx, ridx,
          ra0, rb0, ra1, rb1, ro, a_sp, b_sp, sa0, sb0, sa1, sb1):
        sid = lax.axis_index("s")
        wid = sid * NC + lax.axis_index("c")
        pltpu.sync_copy(a_hbm.at[pl.ds(sid * NTS, NTS)],
                        a_sp.at[pl.ds(sid * NTS, NTS)])
        pltpu.sync_copy(b_hbm.at[pl.ds(sid * NTS, NTS)],
                        b_sp.at[pl.ds(sid * NTS, NTS)])
        pltpu.sync_copy(s_hbm.at[wid], sidx)
        pltpu.sync_copy(r_hbm.at[wid], ridx)
        plsc.subcore_barrier()
        pbase = wid * (EPW // PK)
        cp = CH // PK
        slots = ((ra0, rb0, sa0, sb0), (ra1, rb1, sa1, sb1))

        def issue(j, slot):
            ra, rb, sa, sb = slot
            pltpu.async_copy(a_sp.at[sidx.at[j]], ra, sa)
            pltpu.async_copy(b_sp.at[ridx.at[j]], rb, sb)

        def drain_store(j, slot):
            ra, rb, sa, sb = slot
            pltpu.make_async_copy(a_sp.at[sidx.at[j]], ra, sa).wait()
            pltpu.make_async_copy(b_sp.at[ridx.at[j]], rb, sb).wait()

            @pl.loop(0, cp)
            def _(i2):
                for kk in range(PK):
                    ro[i2, pl.ds(16 * kk, 16)] = (
                        ra[i2 * PK + kk, :] + rb[i2 * PK + kk, :])

            pltpu.sync_copy(ro, g_hbm.at[pl.ds(pbase + j * cp, cp)])

        issue(0, slots[0])

        @pl.loop(0, (NCHUNK - 1) // 2)
        def _(t):
            issue(2 * t + 1, slots[1])
            drain_store(2 * t, slots[0])
            issue(2 * t + 2, slots[0])
            drain_store(2 * t + 1, slots[1])

        drain_store(NCHUNK - 1, slots[0])

    return k(a, b, s2, r2)


def _sc_scatter(ne, s2, r2):
    mesh = plsc.VectorSubcoreMesh(core_axis_name="c", subcore_axis_name="s")

    @functools.partial(
        pl.kernel,
        out_type=(jax.ShapeDtypeStruct((NC, NPAD, 16), _f32),
                  jax.ShapeDtypeStruct((NC, NPAD, 16), _f32)),
        mesh=mesh,
        scratch_types=[
            pltpu.VMEM((NCHUNK, CH), jnp.int32),
            pltpu.VMEM((NCHUNK, CH), jnp.int32),
            pltpu.VMEM((CH, 16), _f32),
            pltpu.VMEM((CH, 16), _f32),
            pltpu.VMEM((128, 16), _f32),
            pltpu.VMEM_SHARED((NPAD, 16), _f32),
            pltpu.VMEM_SHARED((NPAD, 16), _f32),
            pltpu.SemaphoreType.DMA,
            pltpu.SemaphoreType.DMA,
            pltpu.SemaphoreType.DMA,
            pltpu.SemaphoreType.DMA,
        ],
        compiler_params=pltpu.CompilerParams(use_tc_tiling_on_sc=False),
    )
    def k(ne_hbm, s_hbm, r_hbm, outs_hbm, outr_hbm, sidx, ridx, rows0, rows1,
          zbuf, accs, accr, sl0, sl1, sadd_s, sadd_r):
        cid = lax.axis_index("c")
        sid = lax.axis_index("s")
        wid = sid * NC + cid

        @pl.loop(0, 128)
        def _(i):
            zbuf[i, :] = jnp.zeros((16,), _f32)

        @pl.loop(0, NPS // 128)
        def _(t):
            pltpu.sync_copy(zbuf, accs.at[pl.ds(sid * NPS + t * 128, 128)])
            pltpu.sync_copy(zbuf, accr.at[pl.ds(sid * NPS + t * 128, 128)])

        pltpu.sync_copy(s_hbm.at[pl.ds(wid * NCHUNK, NCHUNK)], sidx)
        pltpu.sync_copy(r_hbm.at[pl.ds(wid * NCHUNK, NCHUNK)], ridx)
        plsc.subcore_barrier()
        base = wid * EPW
        slots = ((rows0, sl0), (rows1, sl1))

        def issue(j, slot):
            rows, sl = slot
            pltpu.async_copy(ne_hbm.at[pl.ds(base + j * CH, CH)], rows, sl)

        def drain_add(j, slot):
            rows, sl = slot
            pltpu.make_async_copy(
                ne_hbm.at[pl.ds(base + j * CH, CH)], rows, sl).wait()
            ca = pltpu.async_copy(rows, accs.at[sidx.at[j]], sadd_s, add=True)
            cb = pltpu.async_copy(rows, accr.at[ridx.at[j]], sadd_r, add=True)
            ca.wait()
            cb.wait()

        issue(0, slots[0])

        @pl.loop(0, (NCHUNK - 1) // 2)
        def _(t):
            issue(2 * t + 1, slots[1])
            drain_add(2 * t, slots[0])
            issue(2 * t + 2, slots[0])
            drain_add(2 * t + 1, slots[1])

        drain_add(NCHUNK - 1, slots[0])

        plsc.subcore_barrier()
        pltpu.sync_copy(accs.at[pl.ds(sid * NPS, NPS)],
                        outs_hbm.at[cid, pl.ds(sid * NPS, NPS)])
        pltpu.sync_copy(accr.at[pl.ds(sid * NPS, NPS)],
                        outr_hbm.at[cid, pl.ds(sid * NPS, NPS)])

    return k(ne, s2, r2)


def kernel(nodes, edges, senders, receivers, aux_data, params):
    (we1, be1), (we2, be2) = params['enc_node']
    (wee1, bee1), (wee2, bee2) = params['enc_edge']
    pe = [params['proc_edge_0'], params['proc_edge_1']]
    pn = [params['proc_node_0'], params['proc_node_1']]
    (wd1, bd1), (wd2, bd2) = params['dec_node']

    pe_e = [p[0][0][0:L] for p in pe]
    pe_s = [p[0][0][L:2 * L] for p in pe]
    pe_r = [p[0][0][2 * L:] for p in pe]
    pn_n = [p[0][0][0:L] for p in pn]
    pn_s = [p[0][0][L:2 * L] for p in pn]
    pn_r = [p[0][0][2 * L:] for p in pn]

    sp = senders.reshape(EP // EBLK, PK, EBLK).transpose(0, 2, 1).reshape(-1)
    rp = receivers.reshape(
        EP // EBLK, PK, EBLK).transpose(0, 2, 1).reshape(-1)
    s2 = sp.reshape(E // CH, CH)
    r2 = rp.reshape(E // CH, CH)
    s3 = sp.reshape(NW, NCHUNK, CH)
    r3 = rp.reshape(NW, NCHUNK, CH)
    nodes_p = nodes.reshape(NP, PK * D)
    edges_t = edges.T
    aux_p = aux_data.reshape(NP, PK * D)

    hn_p, a0_p, b0_p = _single(
        _enc_node_body,
        [jax.ShapeDtypeStruct((NP, 128), _f32)] * 3,
        (nodes_p, _bd(we1), _bt(be1), _bd(we2), _bt(be2),
         _bd(pe_s[0]), _bd(pe_r[0])),
    )
    a_p, b_p = a0_p, b0_p
    he_p = None
    for step in range(2):
        g = _sc_gather(a_p.reshape(N, 16), b_p.reshape(N, 16), s3, r3)
        (w1, b1), (w2, b2) = pe[step]
        wb = (_bd(pe_e[step]), _bt(b1), _bd(w2), _bt(b2))
        if step == 0:
            ne_p, he_p = _edge_grid(
                _enc_edge_step_body, 2, (edges_t, g),
                (_bd(wee1), _bt(bee1), _bd(wee2), _bt(bee2)) + wb,
                first_spec=pl.BlockSpec((16, PK * EBLK), lambda i: (0, i)))
        else:
            (ne_p,) = _edge_grid(_edge_last_body, 1, (g, he_p), wb)
        aggs, aggr = _sc_scatter(ne_p.reshape(E, 16), s2, r2)
        (nw1, nb1), (nw2, nb2) = pn[step]
        s_parts = aggs.reshape(NC, NPAD // PK, 128)
        r_parts = aggr.reshape(NC, NPAD // PK, 128)
        if step == 0:
            hn_p, a_p, b_p = _single(
                _node_step_body,
                [jax.ShapeDtypeStruct((NP, 128), _f32)] * 3,
                (hn_p, s_parts, r_parts,
                 _bd(pn_n[0]), _bd(pn_s[0]), _bd(pn_r[0]), _bt(nb1),
                 _bd(nw2), _bt(nb2), _bd(pe_s[1]), _bd(pe_r[1])),
            )
        else:
            (out_p,) = _single(
                _node_dec_body,
                [jax.ShapeDtypeStruct((NP, PK * D), _f32)],
                (hn_p, s_parts, r_parts,
                 _bd(pn_n[1]), _bd(pn_s[1]), _bd(pn_r[1]), _bt(nb1),
                 _bd(nw2), _bt(nb2),
                 _bd(wd1), _bt(bd1), _bd(wd2), _bt(bd2), aux_p),
            )
    return out_p.reshape(N, D)
